# Initial kernel scaffold; baseline (speedup 1.0000x reference)
import jax
import jax.numpy as jnp
from jax.experimental import pallas as pl


def kernel(x, table):
    raise NotImplementedError("write your pallas kernel here")


# speedup vs baseline: 1.5636x; 1.5636x over previous
import functools

import jax
import jax.numpy as jnp
from jax import lax
from jax.experimental import pallas as pl
from jax.experimental.pallas import tpu as pltpu
from jax.experimental.pallas import tpu_sc as plsc

EMBED_DIM = 32
NUM_CORES = 2
NUM_SUBCORES = 16
NW = NUM_CORES * NUM_SUBCORES
IDX_LANE = 128


@functools.lru_cache(maxsize=None)
def _make_gather(n_vecs_total: int):
    vecs_per_w = n_vecs_total // NW
    VPG = 8
    n_groups = vecs_per_w // VPG
    G_ROWS = VPG * IDX_LANE
    B = n_vecs_total * IDX_LANE

    mesh = plsc.VectorSubcoreMesh(
        core_axis_name="c", subcore_axis_name="s",
        num_cores=NUM_CORES, num_subcores=NUM_SUBCORES)

    @functools.partial(
        pl.kernel,
        mesh=mesh,
        out_type=jax.ShapeDtypeStruct((B, EMBED_DIM), jnp.float32),
        scratch_types=[
            pltpu.VMEM((vecs_per_w, IDX_LANE), jnp.int32),
            pltpu.VMEM((G_ROWS, EMBED_DIM), jnp.float32),
            pltpu.SemaphoreType.DMA,
        ],
        compiler_params=pltpu.CompilerParams(use_tc_tiling_on_sc=False),
    )
    def gather_kernel(idx_hbm, table_hbm, out_hbm, idx_v, rows_v, sem):
        wid = lax.axis_index("s") * NUM_CORES + lax.axis_index("c")
        vbase = wid * vecs_per_w
        rbase = wid * vecs_per_w * IDX_LANE
        pltpu.sync_copy(idx_hbm.at[pl.ds(vbase, vecs_per_w)], idx_v)

        @pl.loop(0, n_groups)
        def _group(g):
            goff = g * VPG
            descs = [
                pltpu.async_copy(
                    table_hbm.at[idx_v.at[goff + j]],
                    rows_v.at[pl.ds(j * IDX_LANE, IDX_LANE)],
                    sem)
                for j in range(VPG)
            ]
            for d in descs:
                d.wait()
            pltpu.sync_copy(
                rows_v, out_hbm.at[pl.ds(rbase + g * G_ROWS, G_ROWS)])

    return gather_kernel


def kernel(x, table):
    orig_shape = x.shape
    idx = x.reshape(-1).astype(jnp.int32)
    n = idx.shape[0]
    assert n % (NW * IDX_LANE * 8) == 0, n
    idx2d = idx.reshape(-1, IDX_LANE)
    out = _make_gather(idx2d.shape[0])(idx2d, table)
    return out.reshape(*orig_shape, EMBED_DIM)


# speedup vs baseline: 1.5773x; 1.0087x over previous
import functools

import jax
import jax.numpy as jnp
from jax import lax
from jax.experimental import pallas as pl
from jax.experimental.pallas import tpu as pltpu
from jax.experimental.pallas import tpu_sc as plsc

EMBED_DIM = 32
NUM_CORES = 2
NUM_SUBCORES = 16
NW = NUM_CORES * NUM_SUBCORES
IDX_LANE = 128
VPG = 13


@functools.lru_cache(maxsize=None)
def _make_gather(n_vecs_total: int):
    vecs_per_w = n_vecs_total // NW
    n_groups = vecs_per_w // VPG
    G_ROWS = VPG * IDX_LANE
    B = n_vecs_total * IDX_LANE

    mesh = plsc.VectorSubcoreMesh(
        core_axis_name="c", subcore_axis_name="s",
        num_cores=NUM_CORES, num_subcores=NUM_SUBCORES)

    @functools.partial(
        pl.kernel,
        mesh=mesh,
        out_type=jax.ShapeDtypeStruct((B, EMBED_DIM), jnp.float32),
        scratch_types=[
            pltpu.VMEM((vecs_per_w, IDX_LANE), jnp.int32),
            pltpu.VMEM((2, G_ROWS, EMBED_DIM), jnp.float32),
            pltpu.SemaphoreType.DMA,
            pltpu.SemaphoreType.DMA,
        ],
        compiler_params=pltpu.CompilerParams(use_tc_tiling_on_sc=False),
    )
    def gather_kernel(idx_hbm, table_hbm, out_hbm, idx_v, rows_v, gsem, osem):
        wid = lax.axis_index("s") * NUM_CORES + lax.axis_index("c")
        vbase = wid * vecs_per_w
        rbase = wid * vecs_per_w * IDX_LANE
        pltpu.sync_copy(idx_hbm.at[pl.ds(vbase, vecs_per_w)], idx_v)

        def issue_gathers(g, slot):
            goff = g * VPG
            for j in range(VPG):
                pltpu.async_copy(
                    table_hbm.at[idx_v.at[goff + j]],
                    rows_v.at[slot].at[pl.ds(j * IDX_LANE, IDX_LANE)],
                    gsem)

        def drain_gathers(slot):
            for j in range(VPG):
                pltpu.make_async_copy(
                    table_hbm.at[idx_v.at[j]],
                    rows_v.at[slot].at[pl.ds(j * IDX_LANE, IDX_LANE)],
                    gsem).wait()

        def out_slice(g):
            return out_hbm.at[pl.ds(rbase + g * G_ROWS, G_ROWS)]

        issue_gathers(0, 0)

        @pl.loop(0, n_groups)
        def _group(g):
            slot = lax.rem(g, 2)
            nslot = lax.rem(g + 1, 2)

            @pl.when(g >= 1)
            def _wait_prev_out():
                pltpu.make_async_copy(rows_v.at[nslot], out_slice(0),
                                      osem).wait()

            @pl.when(g + 1 < n_groups)
            def _issue_next():
                issue_gathers(g + 1, nslot)

            drain_gathers(slot)
            pltpu.async_copy(rows_v.at[slot], out_slice(g), osem)

        pltpu.make_async_copy(rows_v.at[0], out_slice(0), osem).wait()

    return gather_kernel


def kernel(x, table):
    orig_shape = x.shape
    idx = x.reshape(-1).astype(jnp.int32)
    n = idx.shape[0]
    assert n % (NW * IDX_LANE * VPG) == 0, n
    idx2d = idx.reshape(-1, IDX_LANE)
    out = _make_gather(idx2d.shape[0])(idx2d, table)
    return out.reshape(*orig_shape, EMBED_DIM)
